# baseline (speedup 1.0000x reference)
import functools

import jax
import jax.numpy as jnp
from jax import lax
from jax.experimental import pallas as pl
from jax.experimental.pallas import tpu as pltpu
from jax.experimental.pallas import tpu_sc as plsc

D = 32
NC = 2
NS = 16
NW = NC * NS
SUB = 512
TQ = 4


@jax.jit
def _gather_native(xp, weight):
    n_sub = xp.shape[0] // SUB
    per_w = n_sub // NW
    n_pairs = per_w // 2
    mesh = plsc.VectorSubcoreMesh(core_axis_name="c", subcore_axis_name="s")

    @functools.partial(
        pl.kernel,
        mesh=mesh,
        out_type=jax.ShapeDtypeStruct((200, 4, 128, 8, 128), jnp.float32),
        scratch_types=[
            pltpu.VMEM((2, SUB), jnp.int32),
            pltpu.VMEM((2, SUB, D), jnp.float32),
            pltpu.VMEM((2, 4, 4, 8, 128), jnp.float32),
            [pltpu.SemaphoreType.DMA] * 2,
            [pltpu.SemaphoreType.DMA] * 2,
            [pltpu.SemaphoreType.DMA] * 2,
        ],
        compiler_params=pltpu.CompilerParams(
            use_tc_tiling_on_sc=False, needs_layout_passes=False
        ),
    )
    def k(xp_hbm, table_hbm, out_hbm, idx_v, rows_v, dst_v, sem_i, sem_g, sem_o):
        wid = lax.axis_index("s") * NC + lax.axis_index("c")
        m0 = wid * per_w
        iota16 = lax.iota(jnp.int32, 16)

        def idx_start(n, b):
            pltpu.async_copy(
                xp_hbm.at[pl.ds((m0 + n) * SUB, SUB)], idx_v.at[b], sem_i[b]
            )

        def idx_wait(b):
            pltpu.make_async_copy(
                xp_hbm.at[pl.ds(0, SUB)], idx_v.at[b], sem_i[b]
            ).wait()

        def gather_start(b):
            pltpu.async_copy(table_hbm.at[idx_v.at[b]], rows_v.at[b], sem_g[b])

        def gather_wait(b):
            pltpu.make_async_copy(
                table_hbm.at[idx_v.at[b]], rows_v.at[b], sem_g[b]
            ).wait()

        def transpose(b):
            rows = rows_v.at[b]

            def dbody(d0, carry):
                cidx = (d0 + iota16) & (D - 1)
                dtrv = cidx >> 3
                ddrv = cidx & 7
                vs = []
                for g in range(32):
                    ridx = iota16 + g * 16
                    vs.append(plsc.load_gather(rows, [ridx, cidx]))
                for g in range(32):
                    plsc.store_scatter(
                        dst_v.at[b],
                        [dtrv, jnp.full((16,), g // 8, jnp.int32), ddrv,
                         iota16 + (g % 8) * 16],
                        vs[g],
                    )
                return carry

            lax.fori_loop(0, D, dbody, 0)

        def out_start(n, b):
            m = m0 + n
            t = m // 32
            btc0 = (m % 32) * 4
            pltpu.async_copy(
                dst_v.at[b],
                out_hbm.at[t, :, pl.ds(btc0, 4)],
                sem_o[b],
            )

        def out_wait(b):
            pltpu.make_async_copy(
                dst_v.at[b], out_hbm.at[0, :, pl.ds(0, 4)], sem_o[b]
            ).wait()

        idx_start(0, 0)
        idx_start(1, 1)
        idx_wait(0)
        gather_start(0)

        def pair(p, carry):
            for nb in range(2):
                n = 2 * p + nb
                other = 1 - nb

                gather_wait(nb)

                @pl.when(n + 2 < per_w)
                def _():
                    idx_start(n + 2, nb)

                @pl.when(n + 1 < per_w)
                def _():
                    idx_wait(other)
                    gather_start(other)

                @pl.when(n >= 2)
                def _():
                    out_wait(nb)

                transpose(nb)
                out_start(n, nb)
            return carry

        lax.fori_loop(0, n_pairs, pair, 0)

        out_wait(0)
        out_wait(1)

    return k(xp, weight)


def kernel(x, weight):
    rows, cols = x.shape
    xp = x.astype(jnp.int32).transpose(1, 0).reshape(rows * cols)
    out5 = _gather_native(xp, weight)
    return out5.transpose(2, 4, 0, 1, 3).reshape(rows, cols, D)
